# Initial kernel scaffold; baseline (speedup 1.0000x reference)
import jax
import jax.numpy as jnp
from jax.experimental import pallas as pl


def kernel(indices, word_embeddings):
    raise NotImplementedError("write your pallas kernel here")


# speedup vs baseline: 2.5038x; 2.5038x over previous
import functools

import jax
import jax.numpy as jnp
from jax import lax
from jax.experimental import pallas as pl
from jax.experimental.pallas import tpu as pltpu
from jax.experimental.pallas import tpu_sc as plsc

VOCAB = 100
HIDDEN = 128

NUM_CORES = 2
NUM_SUBCORES = 16
NUM_WORKERS = NUM_CORES * NUM_SUBCORES

CHUNK = 128


def _emb_kernel(n_total, idx_hbm, tab_hbm, out_hbm, idx_v, rows_v, gsem):
    per_w = n_total // NUM_WORKERS
    n_chunks = per_w // CHUNK
    wid = lax.axis_index("s") * NUM_CORES + lax.axis_index("c")
    base = wid * per_w
    pltpu.sync_copy(idx_hbm.at[pl.ds(base, per_w)], idx_v)

    def body(i):
        off = i * CHUNK
        pltpu.async_copy(
            tab_hbm.at[idx_v.at[pl.ds(off, CHUNK)]], rows_v, gsem
        ).wait()
        pltpu.sync_copy(rows_v, out_hbm.at[pl.ds(base + off, CHUNK)])

    pl.loop(0, n_chunks)(body)


def kernel(indices, word_embeddings):
    batch, hist = indices.shape
    n_total = batch * hist
    idx_flat = indices.reshape(n_total).astype(jnp.int32)

    mesh = plsc.VectorSubcoreMesh(
        core_axis_name="c", subcore_axis_name="s",
        num_cores=NUM_CORES, num_subcores=NUM_SUBCORES,
    )
    per_w = n_total // NUM_WORKERS

    emb = functools.partial(
        pl.kernel,
        out_type=jax.ShapeDtypeStruct((n_total, HIDDEN), jnp.float32),
        mesh=mesh,
        scratch_types=[
            pltpu.VMEM((per_w,), jnp.int32),
            pltpu.VMEM((CHUNK, HIDDEN), jnp.float32),
            pltpu.SemaphoreType.DMA,
        ],
    )(functools.partial(_emb_kernel, n_total))

    out = emb(idx_flat, word_embeddings)
    return out.reshape(batch, hist, HIDDEN)


# speedup vs baseline: 2.5427x; 1.0156x over previous
import functools

import jax
import jax.numpy as jnp
from jax import lax
from jax.experimental import pallas as pl
from jax.experimental.pallas import tpu as pltpu
from jax.experimental.pallas import tpu_sc as plsc

VOCAB = 100
HIDDEN = 128

NUM_CORES = 2
NUM_SUBCORES = 16
NUM_WORKERS = NUM_CORES * NUM_SUBCORES

CHUNK = 128
NBUF = 4


def _emb_kernel(n_total, idx_hbm, tab_hbm, out_hbm, idx_v, rows_v, *sems):
    gsem, ssem = sems[:NBUF], sems[NBUF:]
    per_w = n_total // NUM_WORKERS
    n_chunks = per_w // CHUNK
    wid = lax.axis_index("s") * NUM_CORES + lax.axis_index("c")
    base = wid * per_w
    pltpu.sync_copy(idx_hbm.at[pl.ds(base, per_w)], idx_v)

    def gather(b, c):
        return pltpu.make_async_copy(
            tab_hbm.at[idx_v.at[pl.ds(c * CHUNK, CHUNK)]],
            rows_v.at[b], gsem[b])

    def scatter(b, c):
        return pltpu.make_async_copy(
            rows_v.at[b], out_hbm.at[pl.ds(base + c * CHUNK, CHUNK)],
            ssem[b])

    for b in range(NBUF):
        gather(b, b).start()

    def outer(j):
        i0 = j * NBUF
        for b in range(NBUF):
            c = i0 + b
            gather(b, c).wait()
            scatter(b, c).start()
            scatter(b, c).wait()
            gather(b, c + NBUF).start()

    pl.loop(0, (n_chunks - NBUF) // NBUF)(outer)

    for b in range(NBUF):
        c = n_chunks - NBUF + b
        gather(b, c).wait()
        scatter(b, c).start()
    for b in range(NBUF):
        scatter(b, n_chunks - NBUF + b).wait()


def kernel(indices, word_embeddings):
    batch, hist = indices.shape
    n_total = batch * hist
    idx_flat = indices.reshape(n_total).astype(jnp.int32)

    mesh = plsc.VectorSubcoreMesh(
        core_axis_name="c", subcore_axis_name="s",
        num_cores=NUM_CORES, num_subcores=NUM_SUBCORES,
    )
    per_w = n_total // NUM_WORKERS

    emb = functools.partial(
        pl.kernel,
        out_type=jax.ShapeDtypeStruct((n_total, HIDDEN), jnp.float32),
        mesh=mesh,
        scratch_types=[
            pltpu.VMEM((per_w,), jnp.int32),
            pltpu.VMEM((NBUF, CHUNK, HIDDEN), jnp.float32),
        ] + [pltpu.SemaphoreType.DMA] * (2 * NBUF),
    )(functools.partial(_emb_kernel, n_total))

    out = emb(idx_flat, word_embeddings)
    return out.reshape(batch, hist, HIDDEN)
